# baseline (speedup 1.0000x reference)
import dataclasses
import functools

import jax
import jax.numpy as jnp
from jax import lax
from jax.experimental import pallas as pl
from jax.experimental.pallas import tpu as pltpu
from jax.experimental.pallas import tpu_sc as plsc

NC = 2
NS = 16
L = 16
ROW = 128


def _ceil_to(x, m):
    return (x + m - 1) // m * m


def _sc_compiler_params(tc_tiling=True):
    cp = pltpu.CompilerParams()
    fields = pltpu.CompilerParams.__dataclass_fields__
    if "needs_layout_passes" in fields:
        cp = dataclasses.replace(cp, needs_layout_passes=False)
    if not tc_tiling and "use_tc_tiling_on_sc" in fields:
        cp = dataclasses.replace(cp, use_tc_tiling_on_sc=False)
    return cp


def _deg_kernel(dst2, ew2, n_pad):
    R = dst2.shape[0]
    G = 8
    rps = R // (NC * NS)
    npsub = n_pad // NS
    mesh = plsc.VectorSubcoreMesh(core_axis_name="c", subcore_axis_name="s")

    @functools.partial(
        pl.kernel,
        out_type=jax.ShapeDtypeStruct((NC * n_pad,), jnp.float32),
        mesh=mesh,
        scratch_types=[
            pltpu.VMEM((G, ROW), jnp.int32),
            pltpu.VMEM((G, ROW), jnp.float32),
            pltpu.VMEM((npsub,), jnp.float32),
            pltpu.VMEM_SHARED((n_pad,), jnp.float32),
        ],
    )
    def k(dst_hbm, ew_hbm, out_hbm, dstb, ewb, stage, deg_sh):
        c = lax.axis_index("c")
        s = lax.axis_index("s")

        @pl.loop(0, npsub // L)
        def _(i):
            stage[pl.ds(i * L, L)] = jnp.zeros((L,), jnp.float32)

        pltpu.sync_copy(stage, deg_sh.at[pl.ds(s * npsub, npsub)])
        plsc.subcore_barrier()

        base = (c * NS + s) * rps

        @pl.loop(0, rps // G)
        def _(gi):
            pltpu.sync_copy(dst_hbm.at[pl.ds(base + gi * G, G)], dstb)
            pltpu.sync_copy(ew_hbm.at[pl.ds(base + gi * G, G)], ewb)
            for j in range(G):
                pltpu.sync_copy(ewb.at[j], deg_sh.at[dstb.at[j]], add=True)

        plsc.subcore_barrier()
        pltpu.sync_copy(deg_sh.at[pl.ds(s * npsub, npsub)], stage)

        pltpu.sync_copy(stage, out_hbm.at[pl.ds(c * n_pad + s * npsub, npsub)])

    return k(dst2, ew2)


def _linear_kernel(x_time, W, dis2d):
    B, N, C = x_time.shape
    BN = 1000
    grid = (B, N // BN)

    def body(x_ref, w_ref, d_ref, y_ref):
        xw = jnp.dot(x_ref[0], w_ref[...], preferred_element_type=jnp.float32)
        y_ref[0] = xw * d_ref[...]

    return pl.pallas_call(
        body,
        grid=grid,
        in_specs=[
            pl.BlockSpec((1, BN, C), lambda b, j: (b, j, 0)),
            pl.BlockSpec((C, W.shape[1]), lambda b, j: (0, 0)),
            pl.BlockSpec((BN, 1), lambda b, j: (j, 0)),
        ],
        out_specs=pl.BlockSpec((1, BN, W.shape[1]), lambda b, j: (b, j, 0)),
        out_shape=jax.ShapeDtypeStruct((B, N, W.shape[1]), jnp.float32),
    )(x_time, W, dis2d)


def _spmm_kernel(y16i, src2, dst2, ew2, n_pad):
    B = y16i.shape[0]
    N = y16i.shape[1]
    R = src2.shape[0]
    G = 8
    rps = R // NS
    ngroups = rps // G
    npsub = n_pad // NS
    nst = N // NS
    mesh = plsc.VectorSubcoreMesh(core_axis_name="c", subcore_axis_name="s")

    @functools.partial(
        pl.kernel,
        out_type=jax.ShapeDtypeStruct((NC, B, n_pad, 64), jnp.float32),
        mesh=mesh,
        scratch_types=[
            pltpu.VMEM((G, ROW), jnp.int32),
            pltpu.VMEM((G, ROW), jnp.int32),
            pltpu.VMEM((G, ROW), jnp.float32),
            pltpu.VMEM((2, ROW, 64), jnp.int32),
            pltpu.VMEM((ROW, 64), jnp.float32),
            pltpu.VMEM_SHARED((n_pad, 64), jnp.int32),
            pltpu.VMEM_SHARED((n_pad, 64), jnp.float32),
            pltpu.SemaphoreType.DMA,
            pltpu.SemaphoreType.DMA,
            pltpu.SemaphoreType.DMA,
        ],
        compiler_params=_sc_compiler_params(tc_tiling=False),
    )
    def k(y_hbm, src_hbm, dst_hbm, ew_hbm, out_hbm,
          srcb, dstb, ewb, rows16, rowsf, ysp, acc_sh, sg0, sg1, ss0):
        c = lax.axis_index("c")
        s = lax.axis_index("s")
        sg = (sg0, sg1)
        himask = jnp.int32(-65536)

        def scale(p, j):
            @plsc.parallel_loop(0, ROW, unroll=4)
            def _(e):
                ev = plsc.load_gather(
                    ewb.at[j], [jnp.full((L,), e, jnp.int32)])
                idx_e = jnp.full((L,), e, jnp.int32)
                for q in range(2):
                    vi = rows16[p, e, pl.ds(c * 32 + q * L, L)]
                    lo = plsc.bitcast(vi << 16, jnp.float32) * ev
                    hi = plsc.bitcast(vi & himask, jnp.float32) * ev
                    base = jnp.arange(L, dtype=jnp.int32) * 2 + q * 32
                    plsc.store_scatter(rowsf, [idx_e, base], lo)
                    plsc.store_scatter(rowsf, [idx_e, base + 1], hi)

        @pl.loop(0, B)
        def _(bi):
            pltpu.sync_copy(
                y_hbm.at[bi].at[pl.ds(s * nst, nst)],
                ysp.at[pl.ds(s * nst, nst)])
            @pl.loop(0, ROW)
            def _(i):
                for j8 in range(64 // L):
                    rowsf[i, pl.ds(j8 * L, L)] = jnp.zeros((L,),
                                                           jnp.float32)

            for k5 in range(npsub // ROW):
                pltpu.sync_copy(
                    rowsf,
                    acc_sh.at[pl.ds(s * npsub + k5 * ROW, ROW)])
            plsc.subcore_barrier()

            @pl.loop(0, ngroups)
            def _(gi):
                rowbase = s * rps + gi * G
                pltpu.sync_copy(src_hbm.at[pl.ds(rowbase, G)], srcb)
                pltpu.sync_copy(dst_hbm.at[pl.ds(rowbase, G)], dstb)
                pltpu.sync_copy(ew_hbm.at[pl.ds(rowbase, G)], ewb)

                def gath(j, p):
                    return pltpu.async_copy(
                        ysp.at[srcb.at[j]], rows16.at[p], sg[p])

                gd = [None] * G
                sd = None
                gd[0] = gath(0, 0)
                for j in range(G):
                    p = j & 1
                    if j < G - 1:
                        gd[j + 1] = gath(j + 1, 1 - p)
                    gd[j].wait()
                    if sd is not None:
                        sd.wait()
                    scale(p, j)
                    sd = pltpu.async_copy(
                        rowsf, acc_sh.at[dstb.at[j]], ss0, add=True)
                sd.wait()

            plsc.subcore_barrier()
            for k5 in range(npsub // ROW):
                off = s * npsub + k5 * ROW
                pltpu.sync_copy(acc_sh.at[pl.ds(off, ROW)], rowsf)
                pltpu.sync_copy(rowsf,
                                out_hbm.at[c].at[bi].at[pl.ds(off, ROW)])
            plsc.subcore_barrier()

    return k(y16i, src2, dst2, ew2)


def _finalize_kernel(acch, y, dis2d, b):
    B, N, C = y.shape
    BN = 1000
    grid = (B, N // BN)
    H = C // 2

    def body(a0_ref, a1_ref, y_ref, d_ref, b_ref, o_ref):
        av = jnp.concatenate([a0_ref[0, 0], a1_ref[0, 0]], axis=-1)
        o_ref[0] = (av + y_ref[0]) * d_ref[...] + b_ref[...]

    return pl.pallas_call(
        body,
        grid=grid,
        in_specs=[
            pl.BlockSpec((1, 1, BN, H), lambda bb, j: (0, bb, j, 0)),
            pl.BlockSpec((1, 1, BN, H), lambda bb, j: (1, bb, j, 0)),
            pl.BlockSpec((1, BN, C), lambda bb, j: (bb, j, 0)),
            pl.BlockSpec((BN, 1), lambda bb, j: (j, 0)),
            pl.BlockSpec((1, C), lambda bb, j: (0, 0)),
        ],
        out_specs=pl.BlockSpec((1, BN, C), lambda bb, j: (bb, j, 0)),
        out_shape=jax.ShapeDtypeStruct((B, N, C), jnp.float32),
    )(acch, acch, y, dis2d, b.reshape(1, C))


def kernel(x_time, edge_index, edge_weight, W, b):
    B, N, C = x_time.shape
    E = edge_weight.shape[0]
    n_pad = _ceil_to(N, NS * ROW)
    e_pad = _ceil_to(E, NS * ROW * 2 * 8)

    src = edge_index[0].astype(jnp.int32)
    dst = edge_index[1].astype(jnp.int32)
    pad = e_pad - E
    src = jnp.pad(src, (0, pad))
    dst = jnp.pad(dst, (0, pad))
    ew = jnp.pad(edge_weight, (0, pad))

    src2 = src.reshape(e_pad // ROW, ROW)
    dst2 = dst.reshape(e_pad // ROW, ROW)
    ew2 = ew.reshape(e_pad // ROW, ROW)

    deg_p = _deg_kernel(dst2, ew2, n_pad)
    deg = deg_p[:N] + deg_p[n_pad:n_pad + N] + 1.0
    dis = jnp.where(deg > 0, lax.rsqrt(jnp.maximum(deg, 1e-12)), 0.0)
    dis2d = dis[:, None]

    y = _linear_kernel(x_time, W, dis2d)
    y16i = jax.lax.bitcast_convert_type(
        y.astype(jnp.bfloat16).reshape(B, N, C // 2, 2), jnp.int32)
    acch = _spmm_kernel(y16i, src2, dst2, ew2, n_pad)
    return _finalize_kernel(acch, y, dis2d, b)


# speedup vs baseline: 1.0854x; 1.0854x over previous
import dataclasses
import functools

import jax
import jax.numpy as jnp
from jax import lax
from jax.experimental import pallas as pl
from jax.experimental.pallas import tpu as pltpu
from jax.experimental.pallas import tpu_sc as plsc

NC = 2
NS = 16
L = 16
ROW = 128


def _ceil_to(x, m):
    return (x + m - 1) // m * m


def _sc_compiler_params(tc_tiling=True):
    cp = pltpu.CompilerParams()
    fields = pltpu.CompilerParams.__dataclass_fields__
    if "needs_layout_passes" in fields:
        cp = dataclasses.replace(cp, needs_layout_passes=False)
    if not tc_tiling and "use_tc_tiling_on_sc" in fields:
        cp = dataclasses.replace(cp, use_tc_tiling_on_sc=False)
    return cp


def _deg_kernel(dst2, ew2, n_pad):
    R = dst2.shape[0]
    G = 8
    rps = R // (NC * NS)
    npsub = n_pad // NS
    mesh = plsc.VectorSubcoreMesh(core_axis_name="c", subcore_axis_name="s")

    @functools.partial(
        pl.kernel,
        out_type=jax.ShapeDtypeStruct((NC * n_pad,), jnp.float32),
        mesh=mesh,
        scratch_types=[
            pltpu.VMEM((G, ROW), jnp.int32),
            pltpu.VMEM((G, ROW), jnp.float32),
            pltpu.VMEM((npsub,), jnp.float32),
            pltpu.VMEM_SHARED((n_pad,), jnp.float32),
        ],
    )
    def k(dst_hbm, ew_hbm, out_hbm, dstb, ewb, stage, deg_sh):
        c = lax.axis_index("c")
        s = lax.axis_index("s")

        @pl.loop(0, npsub // L)
        def _(i):
            stage[pl.ds(i * L, L)] = jnp.zeros((L,), jnp.float32)

        pltpu.sync_copy(stage, deg_sh.at[pl.ds(s * npsub, npsub)])
        plsc.subcore_barrier()

        base = (c * NS + s) * rps

        @pl.loop(0, rps // G)
        def _(gi):
            pltpu.sync_copy(dst_hbm.at[pl.ds(base + gi * G, G)], dstb)
            pltpu.sync_copy(ew_hbm.at[pl.ds(base + gi * G, G)], ewb)
            for j in range(G):
                pltpu.sync_copy(ewb.at[j], deg_sh.at[dstb.at[j]], add=True)

        plsc.subcore_barrier()
        pltpu.sync_copy(deg_sh.at[pl.ds(s * npsub, npsub)], stage)

        pltpu.sync_copy(stage, out_hbm.at[pl.ds(c * n_pad + s * npsub, npsub)])

    return k(dst2, ew2)


def _linear_kernel(x_time, W, dis2d):
    B, N, C = x_time.shape
    BN = 1000
    grid = (B, N // BN)

    def body(x_ref, w_ref, d_ref, y_ref):
        xw = jnp.dot(x_ref[0], w_ref[...], preferred_element_type=jnp.float32)
        y_ref[0] = xw * d_ref[...]

    return pl.pallas_call(
        body,
        grid=grid,
        in_specs=[
            pl.BlockSpec((1, BN, C), lambda b, j: (b, j, 0)),
            pl.BlockSpec((C, W.shape[1]), lambda b, j: (0, 0)),
            pl.BlockSpec((BN, 1), lambda b, j: (j, 0)),
        ],
        out_specs=pl.BlockSpec((1, BN, W.shape[1]), lambda b, j: (b, j, 0)),
        out_shape=jax.ShapeDtypeStruct((B, N, W.shape[1]), jnp.float32),
    )(x_time, W, dis2d)


def _spmm_kernel(yh, src2, dst2, ew2, n_pad):
    B = yh.shape[1]
    N = yh.shape[2]
    R = src2.shape[0]
    G = 8
    rps = R // NS
    ngroups = rps // G
    npsub = n_pad // NS
    nst = N // NS
    mesh = plsc.VectorSubcoreMesh(core_axis_name="c", subcore_axis_name="s")

    @functools.partial(
        pl.kernel,
        out_type=jax.ShapeDtypeStruct((NC, B, n_pad, 64), jnp.float32),
        mesh=mesh,
        scratch_types=[
            pltpu.VMEM((G, ROW), jnp.int32),
            pltpu.VMEM((G, ROW), jnp.int32),
            pltpu.VMEM((G, ROW), jnp.float32),
            pltpu.VMEM((2, ROW, 32), jnp.int32),
            pltpu.VMEM((ROW, 64), jnp.float32),
            pltpu.VMEM_SHARED((n_pad, 32), jnp.int32),
            pltpu.VMEM_SHARED((n_pad, 64), jnp.float32),
            pltpu.SemaphoreType.DMA,
            pltpu.SemaphoreType.DMA,
            pltpu.SemaphoreType.DMA,
        ],
        compiler_params=_sc_compiler_params(tc_tiling=False),
    )
    def k(y_hbm, src_hbm, dst_hbm, ew_hbm, out_hbm,
          srcb, dstb, ewb, rows16, rowsf, ysp, acc_sh, sg0, sg1, ss0):
        c = lax.axis_index("c")
        s = lax.axis_index("s")
        sg = (sg0, sg1)
        himask = jnp.int32(-65536)

        def scale(p, j):
            @plsc.parallel_loop(0, ROW, unroll=4)
            def _(e):
                ev = plsc.load_gather(
                    ewb.at[j], [jnp.full((L,), e, jnp.int32)])
                idx_e = jnp.full((L,), e, jnp.int32)
                for q in range(2):
                    vi = rows16[p, e, pl.ds(q * L, L)]
                    lo = plsc.bitcast(vi << 16, jnp.float32) * ev
                    hi = plsc.bitcast(vi & himask, jnp.float32) * ev
                    base = jnp.arange(L, dtype=jnp.int32) * 2 + q * 32
                    plsc.store_scatter(rowsf, [idx_e, base], lo)
                    plsc.store_scatter(rowsf, [idx_e, base + 1], hi)

        @pl.loop(0, B)
        def _(bi):
            pltpu.sync_copy(
                y_hbm.at[c].at[bi].at[pl.ds(s * nst, nst)],
                ysp.at[pl.ds(s * nst, nst)])
            @pl.loop(0, ROW)
            def _(i):
                for j8 in range(64 // L):
                    rowsf[i, pl.ds(j8 * L, L)] = jnp.zeros((L,),
                                                           jnp.float32)

            for k5 in range(npsub // ROW):
                pltpu.sync_copy(
                    rowsf,
                    acc_sh.at[pl.ds(s * npsub + k5 * ROW, ROW)])
            plsc.subcore_barrier()

            @pl.loop(0, ngroups)
            def _(gi):
                rowbase = s * rps + gi * G
                pltpu.sync_copy(src_hbm.at[pl.ds(rowbase, G)], srcb)
                pltpu.sync_copy(dst_hbm.at[pl.ds(rowbase, G)], dstb)
                pltpu.sync_copy(ew_hbm.at[pl.ds(rowbase, G)], ewb)

                def gath(j, p):
                    return pltpu.async_copy(
                        ysp.at[srcb.at[j]], rows16.at[p], sg[p])

                gd = [None] * G
                sd = None
                gd[0] = gath(0, 0)
                for j in range(G):
                    p = j & 1
                    if j < G - 1:
                        gd[j + 1] = gath(j + 1, 1 - p)
                    gd[j].wait()
                    if sd is not None:
                        sd.wait()
                    scale(p, j)
                    sd = pltpu.async_copy(
                        rowsf, acc_sh.at[dstb.at[j]], ss0, add=True)
                sd.wait()

            plsc.subcore_barrier()
            for k5 in range(npsub // ROW):
                off = s * npsub + k5 * ROW
                pltpu.sync_copy(acc_sh.at[pl.ds(off, ROW)], rowsf)
                pltpu.sync_copy(rowsf,
                                out_hbm.at[c].at[bi].at[pl.ds(off, ROW)])
            plsc.subcore_barrier()

    return k(yh, src2, dst2, ew2)


def _finalize_kernel(acch, y, dis2d, b):
    B, N, C = y.shape
    BN = 1000
    grid = (B, N // BN)
    H = C // 2

    def body(a0_ref, a1_ref, y_ref, d_ref, b_ref, o_ref):
        av = jnp.concatenate([a0_ref[0, 0], a1_ref[0, 0]], axis=-1)
        o_ref[0] = (av + y_ref[0]) * d_ref[...] + b_ref[...]

    return pl.pallas_call(
        body,
        grid=grid,
        in_specs=[
            pl.BlockSpec((1, 1, BN, H), lambda bb, j: (0, bb, j, 0)),
            pl.BlockSpec((1, 1, BN, H), lambda bb, j: (1, bb, j, 0)),
            pl.BlockSpec((1, BN, C), lambda bb, j: (bb, j, 0)),
            pl.BlockSpec((BN, 1), lambda bb, j: (j, 0)),
            pl.BlockSpec((1, C), lambda bb, j: (0, 0)),
        ],
        out_specs=pl.BlockSpec((1, BN, C), lambda bb, j: (bb, j, 0)),
        out_shape=jax.ShapeDtypeStruct((B, N, C), jnp.float32),
    )(acch, acch, y, dis2d, b.reshape(1, C))


def kernel(x_time, edge_index, edge_weight, W, b):
    B, N, C = x_time.shape
    E = edge_weight.shape[0]
    n_pad = _ceil_to(N, NS * ROW)
    e_pad = _ceil_to(E, NS * ROW * 2 * 8)

    src = edge_index[0].astype(jnp.int32)
    dst = edge_index[1].astype(jnp.int32)
    pad = e_pad - E
    src = jnp.pad(src, (0, pad))
    dst = jnp.pad(dst, (0, pad))
    ew = jnp.pad(edge_weight, (0, pad))

    src2 = src.reshape(e_pad // ROW, ROW)
    dst2 = dst.reshape(e_pad // ROW, ROW)
    ew2 = ew.reshape(e_pad // ROW, ROW)

    deg_p = _deg_kernel(dst2, ew2, n_pad)
    deg = deg_p[:N] + deg_p[n_pad:n_pad + N] + 1.0
    dis = jnp.where(deg > 0, lax.rsqrt(jnp.maximum(deg, 1e-12)), 0.0)
    dis2d = dis[:, None]

    y = _linear_kernel(x_time, W, dis2d)
    y16i = jax.lax.bitcast_convert_type(
        y.astype(jnp.bfloat16).reshape(B, N, C // 2, 2), jnp.int32)
    yh = jnp.stack([y16i[:, :, :C // 4], y16i[:, :, C // 4:]], axis=0)
    acch = _spmm_kernel(yh, src2, dst2, ew2, n_pad)
    return _finalize_kernel(acch, y, dis2d, b)
